# initial kernel scaffold (unmeasured)
import jax
import jax.numpy as jnp
from jax import lax
from jax.experimental import pallas as pl
from jax.experimental.pallas import tpu as pltpu


def kernel(
    x,
):
    def body(*refs):
        pass

    out_shape = jax.ShapeDtypeStruct(..., jnp.float32)
    return pl.pallas_call(body, out_shape=out_shape)(...)


# baseline (device time: 43791 ns/iter reference)
import jax
import jax.numpy as jnp
from jax import lax
from jax.experimental import pallas as pl
from jax.experimental.pallas import tpu as pltpu

Z = 4


def kernel(x):
    m_per, n_full = x.shape
    n_per = n_full // Z

    def body(x_ref, out_ref, xbf_ref, send_sems, recv_sems):
        my_x = lax.axis_index("x")
        my_y = lax.axis_index("y")
        my_z = lax.axis_index("z")

        barrier_sem = pltpu.get_barrier_semaphore()
        for k in range(1, Z):
            pl.semaphore_signal(
                barrier_sem, inc=1,
                device_id=(my_x, my_y, (my_z + k) % Z),
                device_id_type=pl.DeviceIdType.MESH,
            )
        pl.semaphore_wait(barrier_sem, Z - 1)

        xbf_ref[...] = x_ref[...].astype(jnp.bfloat16)

        rdmas = []
        for k in range(1, Z):
            w = (my_z + k) % Z
            rdma = pltpu.make_async_remote_copy(
                src_ref=xbf_ref.at[:, pl.ds(w * n_per, n_per)],
                dst_ref=out_ref.at[pl.ds(my_z * m_per, m_per), :],
                send_sem=send_sems.at[k - 1],
                recv_sem=recv_sems.at[k - 1],
                device_id=(my_x, my_y, w),
                device_id_type=pl.DeviceIdType.MESH,
            )
            rdma.start()
            rdmas.append(rdma)

        out_ref[pl.ds(my_z * m_per, m_per), :] = xbf_ref[
            :, pl.ds(my_z * n_per, n_per)
        ]

        for rdma in rdmas:
            rdma.wait()

    return pl.pallas_call(
        body,
        out_shape=jax.ShapeDtypeStruct((Z * m_per, n_per), jnp.bfloat16),
        in_specs=[pl.BlockSpec(memory_space=pltpu.VMEM)],
        out_specs=pl.BlockSpec(memory_space=pltpu.VMEM),
        scratch_shapes=[
            pltpu.VMEM((m_per, n_full), jnp.bfloat16),
            pltpu.SemaphoreType.DMA((Z - 1,)),
            pltpu.SemaphoreType.DMA((Z - 1,)),
        ],
        compiler_params=pltpu.CompilerParams(collective_id=0),
    )(x)
